# baseline (speedup 1.0000x reference)
import functools

import jax
import jax.numpy as jnp
from jax import lax
from jax.experimental import pallas as pl
from jax.experimental.pallas import tpu as pltpu
from jax.experimental.pallas import tpu_sc as plsc

N_OP, N_M, N_AGV = 10000, 1000, 500
C_OP, C_M, C_AGV = 128, 64, 64
E_OP, E_PROC, E_MA = 320000, 320000, 16000
D_EDGE = 16

NC, NS, L = 2, 16, 16
K_BIG = 40
K_SMALL = 32
IB = 8
NSLOT_G = 2
NSLOT_O = 2
RING = 2

f32 = jnp.float32


def _mm_kernel(x_ref, w_ref, b_ref, o_ref):
    o_ref[...] = (
        jnp.dot(x_ref[...], w_ref[...], preferred_element_type=f32)
        + b_ref[...]
    )


def _matmul_bias(x, w, b, block_m):
    m, k = x.shape
    n = w.shape[1]
    grid = m // block_m
    assert block_m * grid == m
    return pl.pallas_call(
        _mm_kernel,
        grid=(grid,),
        in_specs=[
            pl.BlockSpec((block_m, k), lambda i: (i, 0)),
            pl.BlockSpec((k, n), lambda i: (0, 0)),
            pl.BlockSpec((1, n), lambda i: (0, 0)),
        ],
        out_specs=pl.BlockSpec((block_m, n), lambda i: (i, 0)),
        out_shape=jax.ShapeDtypeStruct((m, n), f32),
    )(x, w, b.reshape(1, n))


def _zero_fill(buf, rows, width):
    z = jnp.zeros((L,), f32)

    def body(e, _):
        for j in range(width // L):
            buf[e, pl.ds(L * j, L)] = z
        return _

    lax.fori_loop(0, rows, body, None)


def _run_chunks(eidx, xl_hbm, xr_hbm, ea_hbm, att_vs, acc, c_dim, ke,
                k0, nk, total,
                sidx_ib, didx_ib, xl_s, xr_s, ea_s, out_s, sems, sems_o,
                lane0):
    nj = c_dim // L

    def blk(i):
        return jnp.minimum(k0 + (i // IB) * IB, total - IB)

    def fetch_blk(i):
        r0 = ((i // IB) % 2) * IB
        b0 = blk(i)
        pltpu.sync_copy(eidx.at[0, pl.ds(b0, IB)],
                        sidx_ib.at[pl.ds(r0, IB)])
        pltpu.sync_copy(eidx.at[1, pl.ds(b0, IB)],
                        didx_ib.at[pl.ds(r0, IB)])

    def idx_row(ib, i):
        return ib.at[((i // IB) % 2) * IB + (k0 + i - blk(i))]

    def gathers(i, sl):
        pltpu.async_copy(xl_hbm.at[idx_row(sidx_ib, i)], xl_s[sl], sems[sl])
        pltpu.async_copy(xr_hbm.at[idx_row(didx_ib, i)], xr_s[sl], sems[sl])
        if ea_hbm is not None:
            pltpu.async_copy(ea_hbm.at[pl.ds((k0 + i) * ke, ke)],
                             ea_s[sl], sems[sl])

    def drain(i, sl):
        pltpu.make_async_copy(
            xl_hbm.at[idx_row(sidx_ib, i)], xl_s[sl], sems[sl]).wait()
        pltpu.make_async_copy(
            xr_hbm.at[idx_row(didx_ib, i)], xr_s[sl], sems[sl]).wait()
        if ea_hbm is not None:
            pltpu.make_async_copy(
                ea_hbm.at[pl.ds((k0 + i) * ke, ke)], ea_s[sl],
                sems[sl]).wait()

    def scatter(i, sl):
        pltpu.async_copy(out_s[sl], acc.at[idx_row(didx_ib, i)],
                         sems_o[sl], add=True)

    def drain_scatter(i, sl):
        pltpu.make_async_copy(out_s[sl], acc.at[idx_row(didx_ib, i)],
                              sems_o[sl]).wait()

    def compute(sl, so):
        xl_b, xr_b = xl_s[sl], xr_s[sl]
        out_b = out_s[so]
        ea_b = ea_s[sl] if ea_hbm is not None else None

        def edge_body(e2, _):
            for eo in range(2):
                e = 2 * e2 + eo
                p = jnp.zeros((L,), f32)
                rows = []
                for j in range(nj):
                    v = xl_b[e, pl.ds(L * j, L)]
                    t = v + xr_b[e, pl.ds(L * j, L)]
                    if ea_b is not None:
                        t = t + ea_b[e, pl.ds(L * j, L)]
                    t = jnp.maximum(t, 0.2 * t)
                    p = p + att_vs[j] * t
                    rows.append(v)
                s = jnp.sum(p)
                ex = jnp.exp(lax.broadcast_in_dim(s, (L,), ()))
                for j in range(nj):
                    out_b[e, pl.ds(L * j, L)] = ex * rows[j]
                out_b[e, pl.ds(c_dim, L)] = ex * lane0
            return _

        lax.fori_loop(0, ke // 2, edge_body, None)

    fetch_blk(0)
    for d in range(NSLOT_G - 1):
        gathers(d, d)

    def ring_body(kr, _):
        for b in range(RING):
            i = RING * kr + b

            def step():
                nxt = i + NSLOT_G - 1

                @pl.when(jnp.logical_and(nxt < nk, nxt % IB == 0))
                def _():
                    fetch_blk(nxt)

                @pl.when(nxt < nk)
                def _():
                    gathers(nxt, (b + NSLOT_G - 1) % NSLOT_G)

                drain(i, b % NSLOT_G)

                @pl.when(i >= NSLOT_O)
                def _():
                    drain_scatter(i - NSLOT_O, b % NSLOT_O)

                compute(b % NSLOT_G, b % NSLOT_O)
                scatter(i, b % NSLOT_O)

            if b == 0:
                step()
            else:
                pl.when(i < nk)(step)
        return _

    lax.fori_loop(0, (nk + RING - 1) // RING, ring_body, None)

    for sl in range(NSLOT_O):
        pend = (nk - 1) - (nk - 1 - sl) % NSLOT_O
        drain_scatter(pend, sl)


def _sc_edge_kernel(
    xl_pred, xr_pred, xl_succ, xr_succ, xl_proc, xr_proc, ea_proc,
    xl_ma, xr_ma, att_pred, att_succ, att_proc, att_ma,
    ei_pred, ei_succ, ei_proc, ei_ma,
    out_big, out_proc, out_ma,
    acc_big, acc_proc, acc_ma,
):
    c = lax.axis_index("c")
    s = lax.axis_index("s")
    wid = c * NS + s

    lane0 = (lax.broadcasted_iota(jnp.int32, (L,), 0) == 0).astype(f32)

    def zero_rows_with(out_b, ke, acc, row0, nrows):
        zfull, zrem = divmod(nrows, ke)
        for kk in range(zfull):
            pltpu.sync_copy(out_b, acc.at[pl.ds(row0 + ke * kk, ke)])
        if zrem:
            pltpu.sync_copy(out_b.at[pl.ds(0, zrem)],
                            acc.at[pl.ds(row0 + ke * zfull, zrem)])

    def big_phase(sidx_ib, didx_ib, xl0, xl1, xr0, xr1,
                  out0, out1, att_b,
                  sem0, sem1, semo0, semo1):
        xl_s, xr_s = (xl0, xl1), (xr0, xr1)
        out_s = (out0, out1)
        sems, sems_o = (sem0, sem1), (semo0, semo1)
        _zero_fill(out0, K_BIG, C_OP + L)
        rows = N_OP // NS
        r0 = rows * s
        zero_rows_with(out0, K_BIG, acc_big, r0, rows)
        plsc.subcore_barrier()

        total = E_OP // K_BIG
        base, extra = divmod(total, NS)
        nk = base + (s < extra).astype(jnp.int32)
        k0 = base * s + jnp.minimum(s, extra)

        @pl.when(c == 0)
        def _():
            pltpu.sync_copy(att_pred, att_b)
            att_vs = [att_b[pl.ds(L * j, L)] for j in range(C_OP // L)]
            _run_chunks(ei_pred, xl_pred, xr_pred, None, att_vs, acc_big,
                        C_OP, K_BIG, k0, nk, total,
                        sidx_ib, didx_ib, xl_s, xr_s, None,
                        out_s, sems, sems_o, lane0)

        @pl.when(c == 1)
        def _():
            pltpu.sync_copy(att_succ, att_b)
            att_vs = [att_b[pl.ds(L * j, L)] for j in range(C_OP // L)]
            _run_chunks(ei_succ, xl_succ, xr_succ, None, att_vs, acc_big,
                        C_OP, K_BIG, k0, nk, total,
                        sidx_ib, didx_ib, xl_s, xr_s, None,
                        out_s, sems, sems_o, lane0)

        plsc.subcore_barrier()
        pltpu.sync_copy(acc_big.at[pl.ds(r0, rows)],
                        out_big.at[c, pl.ds(r0, rows)])

    pl.run_scoped(
        big_phase,
        pltpu.VMEM((2 * IB, K_BIG), jnp.int32),
        pltpu.VMEM((2 * IB, K_BIG), jnp.int32),
        pltpu.VMEM((K_BIG, C_OP), f32),
        pltpu.VMEM((K_BIG, C_OP), f32),
        pltpu.VMEM((K_BIG, C_OP), f32),
        pltpu.VMEM((K_BIG, C_OP), f32),
        pltpu.VMEM((K_BIG, C_OP + L), f32),
        pltpu.VMEM((K_BIG, C_OP + L), f32),
        pltpu.VMEM((C_OP,), f32),
        pltpu.SemaphoreType.DMA,
        pltpu.SemaphoreType.DMA,
        pltpu.SemaphoreType.DMA,
        pltpu.SemaphoreType.DMA,
    )

    def small_phase(sidx_ib, didx_ib, xl0, xl1, xr0, xr1,
                    ea0, ea1, out0, out1, att_b,
                    sem0, sem1, semo0, semo1):
        xl_s, xr_s = (xl0, xl1), (xr0, xr1)
        ea_s, out_s = (ea0, ea1), (out0, out1)
        sems, sems_o = (sem0, sem1), (semo0, semo1)
        _zero_fill(out0, K_SMALL, C_M + L)

        @pl.when(s < 8)
        def _():
            zero_rows_with(out0, K_SMALL, acc_proc, 125 * s, 125)

        @pl.when(jnp.logical_and(c == 0, s < 4))
        def _():
            zero_rows_with(out0, K_SMALL, acc_ma, 125 * s, 125)

        plsc.subcore_barrier()

        pltpu.sync_copy(att_proc, att_b)
        att_vs = [att_b[pl.ds(L * j, L)] for j in range(C_M // L)]
        total_p = E_PROC // K_SMALL
        pbase, pextra = divmod(total_p, NC * NS)
        nk = pbase + (wid < pextra).astype(jnp.int32)
        k0 = pbase * wid + jnp.minimum(wid, pextra)
        _run_chunks(ei_proc, xl_proc, xr_proc, ea_proc, att_vs, acc_proc,
                    C_M, K_SMALL, k0, nk, total_p,
                    sidx_ib, didx_ib, xl_s, xr_s, ea_s,
                    out_s, sems, sems_o, lane0)

        @pl.when(c == 0)
        def _():
            pltpu.sync_copy(att_ma, att_b)
            att_vs2 = [att_b[pl.ds(L * j, L)] for j in range(C_M // L)]
            total_m = E_MA // K_SMALL
            mbase, mextra = divmod(total_m, NS)
            nk2 = mbase + (s < mextra).astype(jnp.int32)
            k02 = mbase * s + jnp.minimum(s, mextra)
            _run_chunks(ei_ma, xl_ma, xr_ma, None, att_vs2, acc_ma,
                        C_M, K_SMALL, k02, nk2, total_m,
                        sidx_ib, didx_ib, xl_s, xr_s, None,
                        out_s, sems, sems_o, lane0)

        plsc.subcore_barrier()

        @pl.when(s < 10)
        def _():
            pltpu.sync_copy(acc_proc.at[pl.ds(100 * s, 100)],
                            out_proc.at[c, pl.ds(100 * s, 100)])

        @pl.when(jnp.logical_and(c == 0, s < 10))
        def _():
            pltpu.sync_copy(acc_ma.at[pl.ds(50 * s, 50)],
                            out_ma.at[pl.ds(50 * s, 50)])

    pl.run_scoped(
        small_phase,
        pltpu.VMEM((2 * IB, K_SMALL), jnp.int32),
        pltpu.VMEM((2 * IB, K_SMALL), jnp.int32),
        pltpu.VMEM((K_SMALL, C_M), f32),
        pltpu.VMEM((K_SMALL, C_M), f32),
        pltpu.VMEM((K_SMALL, C_M), f32),
        pltpu.VMEM((K_SMALL, C_M), f32),
        pltpu.VMEM((K_SMALL, C_M), f32),
        pltpu.VMEM((K_SMALL, C_M), f32),
        pltpu.VMEM((K_SMALL, C_M + L), f32),
        pltpu.VMEM((K_SMALL, C_M + L), f32),
        pltpu.VMEM((C_M,), f32),
        pltpu.SemaphoreType.DMA,
        pltpu.SemaphoreType.DMA,
        pltpu.SemaphoreType.DMA,
        pltpu.SemaphoreType.DMA,
    )


_sc_edge_call = functools.partial(
    pl.kernel,
    out_type=(
        jax.ShapeDtypeStruct((NC, N_OP, C_OP + L), f32),
        jax.ShapeDtypeStruct((NC, N_M, C_M + L), f32),
        jax.ShapeDtypeStruct((N_AGV, C_M + L), f32),
    ),
    mesh=plsc.VectorSubcoreMesh(core_axis_name="c", subcore_axis_name="s"),
    compiler_params=pltpu.CompilerParams(
        use_tc_tiling_on_sc=False, needs_layout_passes=False),
    scratch_types=[
        pltpu.VMEM_SHARED((N_OP, C_OP + L), f32),
        pltpu.VMEM_SHARED((N_M, C_M + L), f32),
        pltpu.VMEM_SHARED((N_AGV, C_M + L), f32),
    ],
)(_sc_edge_kernel)


def _finalize_kernel(x_op_ref, x_m_ref, x_agv_ref, big_ref, proc_ref, ma_ref,
                     bp_ref, bs_ref, bm_ref, ba_ref,
                     o_op_ref, o_m_ref, o_agv_ref):
    accp = big_ref[0]
    accs = big_ref[1]
    o_op_ref[...] = (
        x_op_ref[...]
        + accp[:, :C_OP] / (accp[:, C_OP:C_OP + 1] + 1e-16) + bp_ref[...]
        + accs[:, :C_OP] / (accs[:, C_OP:C_OP + 1] + 1e-16) + bs_ref[...]
    )
    pr = proc_ref[0] + proc_ref[1]
    o_m_ref[...] = (
        x_m_ref[...] + pr[:, :C_M] / (pr[:, C_M:C_M + 1] + 1e-16)
        + bm_ref[...]
    )
    ma = ma_ref[...]
    o_agv_ref[...] = (
        x_agv_ref[...] + ma[:, :C_M] / (ma[:, C_M:C_M + 1] + 1e-16)
        + ba_ref[...]
    )


def _finalize(x_op, x_m, x_agv, big, proc, ma, bp, bs, bm, ba):
    return pl.pallas_call(
        _finalize_kernel,
        out_shape=(
            jax.ShapeDtypeStruct((N_OP, C_OP), f32),
            jax.ShapeDtypeStruct((N_M, C_M), f32),
            jax.ShapeDtypeStruct((N_AGV, C_AGV), f32),
        ),
    )(x_op, x_m, x_agv, big, proc, ma,
      bp.reshape(1, -1), bs.reshape(1, -1), bm.reshape(1, -1),
      ba.reshape(1, -1))


@jax.jit
def kernel(x_operation, x_machine, x_AGV, edge_index_pred, edge_index_succ,
           edge_index_proc, edge_index_ma, edge_attr_proc,
           Wl_pred, bl_pred, Wr_pred, br_pred, att_pred, bias_pred,
           Wl_succ, bl_succ, Wr_succ, br_succ, att_succ, bias_succ,
           Wl_proc, bl_proc, Wr_proc, br_proc, att_proc, We_proc, bias_proc,
           Wl_ma, bl_ma, Wr_ma, br_ma, att_ma, bias_ma):
    w_op = jnp.concatenate([Wl_pred, Wr_pred, Wl_succ, Wr_succ, Wl_proc], 1)
    b_op = jnp.concatenate([bl_pred, br_pred, bl_succ, br_succ, bl_proc])
    proj_op = _matmul_bias(x_operation, w_op, b_op, 1000)
    xl_pred = proj_op[:, 0:128]
    xr_pred = proj_op[:, 128:256]
    xl_succ = proj_op[:, 256:384]
    xr_succ = proj_op[:, 384:512]
    xl_proc = proj_op[:, 512:576]

    w_m = jnp.concatenate([Wr_proc, Wl_ma], 1)
    b_m = jnp.concatenate([br_proc, bl_ma])
    proj_m = _matmul_bias(x_machine, w_m, b_m, 1000)
    xr_proc = proj_m[:, 0:64]
    xl_ma = proj_m[:, 64:128]

    xr_ma = _matmul_bias(x_AGV, Wr_ma, br_ma, 500)
    ea_proc = _matmul_bias(edge_attr_proc, We_proc,
                           jnp.zeros((C_M,), f32), 8000)

    out_big, out_proc, out_ma = _sc_edge_call(
        xl_pred, xr_pred, xl_succ, xr_succ, xl_proc, xr_proc, ea_proc,
        xl_ma, xr_ma, att_pred, att_succ, att_proc, att_ma,
        edge_index_pred.reshape(2, E_OP // K_BIG, K_BIG),
        edge_index_succ.reshape(2, E_OP // K_BIG, K_BIG),
        edge_index_proc.reshape(2, E_PROC // K_SMALL, K_SMALL),
        edge_index_ma.reshape(2, E_MA // K_SMALL, K_SMALL))

    return _finalize(x_operation, x_machine, x_AGV, out_big, out_proc, out_ma,
                     bias_pred, bias_succ, bias_proc, bias_ma)


# speedup vs baseline: 1.0058x; 1.0058x over previous
import functools

import jax
import jax.numpy as jnp
from jax import lax
from jax.experimental import pallas as pl
from jax.experimental.pallas import tpu as pltpu
from jax.experimental.pallas import tpu_sc as plsc

N_OP, N_M, N_AGV = 10000, 1000, 500
C_OP, C_M, C_AGV = 128, 64, 64
E_OP, E_PROC, E_MA = 320000, 320000, 16000
D_EDGE = 16

NC, NS, L = 2, 16, 16
K_BIG = 40
K_SMALL = 32
IB = 8
NSLOT_G = 2
NSLOT_O = 2
RING = 2

f32 = jnp.float32


def _mm_kernel(x_ref, w_ref, b_ref, o_ref):
    o_ref[...] = (
        jnp.dot(x_ref[...], w_ref[...], preferred_element_type=f32)
        + b_ref[...]
    )


def _proj_small_kernel(x_m_ref, w_m_ref, b_m_ref, x_a_ref, w_a_ref, b_a_ref,
                       ea_ref, we_ref, o_m_ref, o_a_ref, o_ea_ref):
    @pl.when(pl.program_id(0) == 0)
    def _():
        o_m_ref[...] = (
            jnp.dot(x_m_ref[...], w_m_ref[...], preferred_element_type=f32)
            + b_m_ref[...]
        )
        o_a_ref[...] = (
            jnp.dot(x_a_ref[...], w_a_ref[...], preferred_element_type=f32)
            + b_a_ref[...]
        )
    o_ea_ref[...] = jnp.dot(ea_ref[...], we_ref[...],
                            preferred_element_type=f32)


def _proj_small(x_m, w_m, b_m, x_a, w_a, b_a, ea, we):
    gb = 8000
    return pl.pallas_call(
        _proj_small_kernel,
        grid=(E_PROC // gb,),
        in_specs=[
            pl.BlockSpec((N_M, C_M), lambda i: (0, 0)),
            pl.BlockSpec((C_M, 2 * C_M), lambda i: (0, 0)),
            pl.BlockSpec((1, 2 * C_M), lambda i: (0, 0)),
            pl.BlockSpec((N_AGV, C_M), lambda i: (0, 0)),
            pl.BlockSpec((C_M, C_M), lambda i: (0, 0)),
            pl.BlockSpec((1, C_M), lambda i: (0, 0)),
            pl.BlockSpec((gb, D_EDGE), lambda i: (i, 0)),
            pl.BlockSpec((D_EDGE, C_M), lambda i: (0, 0)),
        ],
        out_specs=[
            pl.BlockSpec((N_M, 2 * C_M), lambda i: (0, 0)),
            pl.BlockSpec((N_AGV, C_M), lambda i: (0, 0)),
            pl.BlockSpec((gb, C_M), lambda i: (i, 0)),
        ],
        out_shape=(
            jax.ShapeDtypeStruct((N_M, 2 * C_M), f32),
            jax.ShapeDtypeStruct((N_AGV, C_M), f32),
            jax.ShapeDtypeStruct((E_PROC, C_M), f32),
        ),
    )(x_m, w_m, b_m.reshape(1, -1), x_a, w_a, b_a.reshape(1, -1), ea, we)


def _matmul_bias(x, w, b, block_m):
    m, k = x.shape
    n = w.shape[1]
    grid = m // block_m
    assert block_m * grid == m
    return pl.pallas_call(
        _mm_kernel,
        grid=(grid,),
        in_specs=[
            pl.BlockSpec((block_m, k), lambda i: (i, 0)),
            pl.BlockSpec((k, n), lambda i: (0, 0)),
            pl.BlockSpec((1, n), lambda i: (0, 0)),
        ],
        out_specs=pl.BlockSpec((block_m, n), lambda i: (i, 0)),
        out_shape=jax.ShapeDtypeStruct((m, n), f32),
    )(x, w, b.reshape(1, n))


def _zero_fill(buf, rows, width):
    z = jnp.zeros((L,), f32)

    def body(e, _):
        for j in range(width // L):
            buf[e, pl.ds(L * j, L)] = z
        return _

    lax.fori_loop(0, rows, body, None)


def _run_chunks(eidx, xl_hbm, xr_hbm, ea_hbm, att_vs, acc, c_dim, ke,
                k0, nk, total,
                sidx_ib, didx_ib, xl_s, xr_s, ea_s, out_s, sems, sems_o,
                lane0):
    nj = c_dim // L

    def blk(i):
        return jnp.minimum(k0 + (i // IB) * IB, total - IB)

    def fetch_blk(i):
        r0 = ((i // IB) % 2) * IB
        b0 = blk(i)
        pltpu.sync_copy(eidx.at[0, pl.ds(b0, IB)],
                        sidx_ib.at[pl.ds(r0, IB)])
        pltpu.sync_copy(eidx.at[1, pl.ds(b0, IB)],
                        didx_ib.at[pl.ds(r0, IB)])

    def idx_row(ib, i):
        return ib.at[((i // IB) % 2) * IB + (k0 + i - blk(i))]

    def gathers(i, sl):
        pltpu.async_copy(xl_hbm.at[idx_row(sidx_ib, i)], xl_s[sl], sems[sl])
        pltpu.async_copy(xr_hbm.at[idx_row(didx_ib, i)], xr_s[sl], sems[sl])
        if ea_hbm is not None:
            pltpu.async_copy(ea_hbm.at[pl.ds((k0 + i) * ke, ke)],
                             ea_s[sl], sems[sl])

    def drain(i, sl):
        pltpu.make_async_copy(
            xl_hbm.at[idx_row(sidx_ib, i)], xl_s[sl], sems[sl]).wait()
        pltpu.make_async_copy(
            xr_hbm.at[idx_row(didx_ib, i)], xr_s[sl], sems[sl]).wait()
        if ea_hbm is not None:
            pltpu.make_async_copy(
                ea_hbm.at[pl.ds((k0 + i) * ke, ke)], ea_s[sl],
                sems[sl]).wait()

    def scatter(i, sl):
        pltpu.async_copy(out_s[sl], acc.at[idx_row(didx_ib, i)],
                         sems_o[sl], add=True)

    def drain_scatter(i, sl):
        pltpu.make_async_copy(out_s[sl], acc.at[idx_row(didx_ib, i)],
                              sems_o[sl]).wait()

    def compute(sl, so):
        xl_b, xr_b = xl_s[sl], xr_s[sl]
        out_b = out_s[so]
        ea_b = ea_s[sl] if ea_hbm is not None else None

        def edge_body(e2, _):
            for eo in range(2):
                e = 2 * e2 + eo
                p = jnp.zeros((L,), f32)
                rows = []
                for j in range(nj):
                    v = xl_b[e, pl.ds(L * j, L)]
                    t = v + xr_b[e, pl.ds(L * j, L)]
                    if ea_b is not None:
                        t = t + ea_b[e, pl.ds(L * j, L)]
                    t = jnp.maximum(t, 0.2 * t)
                    p = p + att_vs[j] * t
                    rows.append(v)
                s = jnp.sum(p)
                ex = jnp.exp(lax.broadcast_in_dim(s, (L,), ()))
                for j in range(nj):
                    out_b[e, pl.ds(L * j, L)] = ex * rows[j]
                out_b[e, pl.ds(c_dim, L)] = ex * lane0
            return _

        lax.fori_loop(0, ke // 2, edge_body, None)

    fetch_blk(0)
    for d in range(NSLOT_G - 1):
        gathers(d, d)

    def ring_body(kr, _):
        for b in range(RING):
            i = RING * kr + b

            def step():
                nxt = i + NSLOT_G - 1

                @pl.when(jnp.logical_and(nxt < nk, nxt % IB == 0))
                def _():
                    fetch_blk(nxt)

                @pl.when(nxt < nk)
                def _():
                    gathers(nxt, (b + NSLOT_G - 1) % NSLOT_G)

                drain(i, b % NSLOT_G)

                @pl.when(i >= NSLOT_O)
                def _():
                    drain_scatter(i - NSLOT_O, b % NSLOT_O)

                compute(b % NSLOT_G, b % NSLOT_O)
                scatter(i, b % NSLOT_O)

            if b == 0:
                step()
            else:
                pl.when(i < nk)(step)
        return _

    lax.fori_loop(0, (nk + RING - 1) // RING, ring_body, None)

    for sl in range(NSLOT_O):
        pend = (nk - 1) - (nk - 1 - sl) % NSLOT_O
        drain_scatter(pend, sl)


def _sc_edge_kernel(
    xl_pred, xr_pred, xl_succ, xr_succ, xl_proc, xr_proc, ea_proc,
    xl_ma, xr_ma, att_pred, att_succ, att_proc, att_ma,
    ei_pred, ei_succ, ei_proc, ei_ma,
    out_big, out_proc, out_ma,
    acc_big, acc_proc, acc_ma,
):
    c = lax.axis_index("c")
    s = lax.axis_index("s")
    wid = c * NS + s

    lane0 = (lax.broadcasted_iota(jnp.int32, (L,), 0) == 0).astype(f32)

    def zero_rows_with(out_b, ke, acc, row0, nrows):
        zfull, zrem = divmod(nrows, ke)
        for kk in range(zfull):
            pltpu.sync_copy(out_b, acc.at[pl.ds(row0 + ke * kk, ke)])
        if zrem:
            pltpu.sync_copy(out_b.at[pl.ds(0, zrem)],
                            acc.at[pl.ds(row0 + ke * zfull, zrem)])

    def big_phase(sidx_ib, didx_ib, xl0, xl1, xr0, xr1,
                  out0, out1, att_b,
                  sem0, sem1, semo0, semo1):
        xl_s, xr_s = (xl0, xl1), (xr0, xr1)
        out_s = (out0, out1)
        sems, sems_o = (sem0, sem1), (semo0, semo1)
        _zero_fill(out0, K_BIG, C_OP + L)
        rows = N_OP // NS
        r0 = rows * s
        zero_rows_with(out0, K_BIG, acc_big, r0, rows)
        plsc.subcore_barrier()

        total = E_OP // K_BIG
        base, extra = divmod(total, NS)
        nk = base + (s < extra).astype(jnp.int32)
        k0 = base * s + jnp.minimum(s, extra)

        @pl.when(c == 0)
        def _():
            pltpu.sync_copy(att_pred, att_b)
            att_vs = [att_b[pl.ds(L * j, L)] for j in range(C_OP // L)]
            _run_chunks(ei_pred, xl_pred, xr_pred, None, att_vs, acc_big,
                        C_OP, K_BIG, k0, nk, total,
                        sidx_ib, didx_ib, xl_s, xr_s, None,
                        out_s, sems, sems_o, lane0)

        @pl.when(c == 1)
        def _():
            pltpu.sync_copy(att_succ, att_b)
            att_vs = [att_b[pl.ds(L * j, L)] for j in range(C_OP // L)]
            _run_chunks(ei_succ, xl_succ, xr_succ, None, att_vs, acc_big,
                        C_OP, K_BIG, k0, nk, total,
                        sidx_ib, didx_ib, xl_s, xr_s, None,
                        out_s, sems, sems_o, lane0)

        plsc.subcore_barrier()
        pltpu.sync_copy(acc_big.at[pl.ds(r0, rows)],
                        out_big.at[c, pl.ds(r0, rows)])

    pl.run_scoped(
        big_phase,
        pltpu.VMEM((2 * IB, K_BIG), jnp.int32),
        pltpu.VMEM((2 * IB, K_BIG), jnp.int32),
        pltpu.VMEM((K_BIG, C_OP), f32),
        pltpu.VMEM((K_BIG, C_OP), f32),
        pltpu.VMEM((K_BIG, C_OP), f32),
        pltpu.VMEM((K_BIG, C_OP), f32),
        pltpu.VMEM((K_BIG, C_OP + L), f32),
        pltpu.VMEM((K_BIG, C_OP + L), f32),
        pltpu.VMEM((C_OP,), f32),
        pltpu.SemaphoreType.DMA,
        pltpu.SemaphoreType.DMA,
        pltpu.SemaphoreType.DMA,
        pltpu.SemaphoreType.DMA,
    )

    def small_phase(sidx_ib, didx_ib, xl0, xl1, xr0, xr1,
                    ea0, ea1, out0, out1, att_b,
                    sem0, sem1, semo0, semo1):
        xl_s, xr_s = (xl0, xl1), (xr0, xr1)
        ea_s, out_s = (ea0, ea1), (out0, out1)
        sems, sems_o = (sem0, sem1), (semo0, semo1)
        _zero_fill(out0, K_SMALL, C_M + L)

        @pl.when(s < 8)
        def _():
            zero_rows_with(out0, K_SMALL, acc_proc, 125 * s, 125)

        @pl.when(jnp.logical_and(c == 0, s < 4))
        def _():
            zero_rows_with(out0, K_SMALL, acc_ma, 125 * s, 125)

        plsc.subcore_barrier()

        pltpu.sync_copy(att_proc, att_b)
        att_vs = [att_b[pl.ds(L * j, L)] for j in range(C_M // L)]
        total_p = E_PROC // K_SMALL
        pbase, pextra = divmod(total_p, NC * NS)
        nk = pbase + (wid < pextra).astype(jnp.int32)
        k0 = pbase * wid + jnp.minimum(wid, pextra)
        _run_chunks(ei_proc, xl_proc, xr_proc, ea_proc, att_vs, acc_proc,
                    C_M, K_SMALL, k0, nk, total_p,
                    sidx_ib, didx_ib, xl_s, xr_s, ea_s,
                    out_s, sems, sems_o, lane0)

        @pl.when(c == 0)
        def _():
            pltpu.sync_copy(att_ma, att_b)
            att_vs2 = [att_b[pl.ds(L * j, L)] for j in range(C_M // L)]
            total_m = E_MA // K_SMALL
            mbase, mextra = divmod(total_m, NS)
            nk2 = mbase + (s < mextra).astype(jnp.int32)
            k02 = mbase * s + jnp.minimum(s, mextra)
            _run_chunks(ei_ma, xl_ma, xr_ma, None, att_vs2, acc_ma,
                        C_M, K_SMALL, k02, nk2, total_m,
                        sidx_ib, didx_ib, xl_s, xr_s, None,
                        out_s, sems, sems_o, lane0)

        plsc.subcore_barrier()

        @pl.when(s < 10)
        def _():
            pltpu.sync_copy(acc_proc.at[pl.ds(100 * s, 100)],
                            out_proc.at[c, pl.ds(100 * s, 100)])

        @pl.when(jnp.logical_and(c == 0, s < 10))
        def _():
            pltpu.sync_copy(acc_ma.at[pl.ds(50 * s, 50)],
                            out_ma.at[pl.ds(50 * s, 50)])

    pl.run_scoped(
        small_phase,
        pltpu.VMEM((2 * IB, K_SMALL), jnp.int32),
        pltpu.VMEM((2 * IB, K_SMALL), jnp.int32),
        pltpu.VMEM((K_SMALL, C_M), f32),
        pltpu.VMEM((K_SMALL, C_M), f32),
        pltpu.VMEM((K_SMALL, C_M), f32),
        pltpu.VMEM((K_SMALL, C_M), f32),
        pltpu.VMEM((K_SMALL, C_M), f32),
        pltpu.VMEM((K_SMALL, C_M), f32),
        pltpu.VMEM((K_SMALL, C_M + L), f32),
        pltpu.VMEM((K_SMALL, C_M + L), f32),
        pltpu.VMEM((C_M,), f32),
        pltpu.SemaphoreType.DMA,
        pltpu.SemaphoreType.DMA,
        pltpu.SemaphoreType.DMA,
        pltpu.SemaphoreType.DMA,
    )


_sc_edge_call = functools.partial(
    pl.kernel,
    out_type=(
        jax.ShapeDtypeStruct((NC, N_OP, C_OP + L), f32),
        jax.ShapeDtypeStruct((NC, N_M, C_M + L), f32),
        jax.ShapeDtypeStruct((N_AGV, C_M + L), f32),
    ),
    mesh=plsc.VectorSubcoreMesh(core_axis_name="c", subcore_axis_name="s"),
    compiler_params=pltpu.CompilerParams(
        use_tc_tiling_on_sc=False, needs_layout_passes=False),
    scratch_types=[
        pltpu.VMEM_SHARED((N_OP, C_OP + L), f32),
        pltpu.VMEM_SHARED((N_M, C_M + L), f32),
        pltpu.VMEM_SHARED((N_AGV, C_M + L), f32),
    ],
)(_sc_edge_kernel)


def _finalize_kernel(x_op_ref, x_m_ref, x_agv_ref, big_ref, proc_ref, ma_ref,
                     bp_ref, bs_ref, bm_ref, ba_ref,
                     o_op_ref, o_m_ref, o_agv_ref):
    accp = big_ref[0]
    accs = big_ref[1]
    o_op_ref[...] = (
        x_op_ref[...]
        + accp[:, :C_OP] / (accp[:, C_OP:C_OP + 1] + 1e-16) + bp_ref[...]
        + accs[:, :C_OP] / (accs[:, C_OP:C_OP + 1] + 1e-16) + bs_ref[...]
    )
    pr = proc_ref[0] + proc_ref[1]
    o_m_ref[...] = (
        x_m_ref[...] + pr[:, :C_M] / (pr[:, C_M:C_M + 1] + 1e-16)
        + bm_ref[...]
    )
    ma = ma_ref[...]
    o_agv_ref[...] = (
        x_agv_ref[...] + ma[:, :C_M] / (ma[:, C_M:C_M + 1] + 1e-16)
        + ba_ref[...]
    )


def _finalize(x_op, x_m, x_agv, big, proc, ma, bp, bs, bm, ba):
    return pl.pallas_call(
        _finalize_kernel,
        out_shape=(
            jax.ShapeDtypeStruct((N_OP, C_OP), f32),
            jax.ShapeDtypeStruct((N_M, C_M), f32),
            jax.ShapeDtypeStruct((N_AGV, C_AGV), f32),
        ),
    )(x_op, x_m, x_agv, big, proc, ma,
      bp.reshape(1, -1), bs.reshape(1, -1), bm.reshape(1, -1),
      ba.reshape(1, -1))


@jax.jit
def kernel(x_operation, x_machine, x_AGV, edge_index_pred, edge_index_succ,
           edge_index_proc, edge_index_ma, edge_attr_proc,
           Wl_pred, bl_pred, Wr_pred, br_pred, att_pred, bias_pred,
           Wl_succ, bl_succ, Wr_succ, br_succ, att_succ, bias_succ,
           Wl_proc, bl_proc, Wr_proc, br_proc, att_proc, We_proc, bias_proc,
           Wl_ma, bl_ma, Wr_ma, br_ma, att_ma, bias_ma):
    w_op = jnp.concatenate([Wl_pred, Wr_pred, Wl_succ, Wr_succ, Wl_proc], 1)
    b_op = jnp.concatenate([bl_pred, br_pred, bl_succ, br_succ, bl_proc])
    proj_op = _matmul_bias(x_operation, w_op, b_op, 1000)
    xl_pred = proj_op[:, 0:128]
    xr_pred = proj_op[:, 128:256]
    xl_succ = proj_op[:, 256:384]
    xr_succ = proj_op[:, 384:512]
    xl_proc = proj_op[:, 512:576]

    w_m = jnp.concatenate([Wr_proc, Wl_ma], 1)
    b_m = jnp.concatenate([br_proc, bl_ma])
    proj_m, xr_ma, ea_proc = _proj_small(
        x_machine, w_m, b_m, x_AGV, Wr_ma, br_ma, edge_attr_proc, We_proc)
    xr_proc = proj_m[:, 0:64]
    xl_ma = proj_m[:, 64:128]

    out_big, out_proc, out_ma = _sc_edge_call(
        xl_pred, xr_pred, xl_succ, xr_succ, xl_proc, xr_proc, ea_proc,
        xl_ma, xr_ma, att_pred, att_succ, att_proc, att_ma,
        edge_index_pred.reshape(2, E_OP // K_BIG, K_BIG),
        edge_index_succ.reshape(2, E_OP // K_BIG, K_BIG),
        edge_index_proc.reshape(2, E_PROC // K_SMALL, K_SMALL),
        edge_index_ma.reshape(2, E_MA // K_SMALL, K_SMALL))

    return _finalize(x_operation, x_machine, x_AGV, out_big, out_proc, out_ma,
                     bias_pred, bias_succ, bias_proc, bias_ma)


# speedup vs baseline: 1.0093x; 1.0035x over previous
import functools

import jax
import jax.numpy as jnp
from jax import lax
from jax.experimental import pallas as pl
from jax.experimental.pallas import tpu as pltpu
from jax.experimental.pallas import tpu_sc as plsc

N_OP, N_M, N_AGV = 10000, 1000, 500
C_OP, C_M, C_AGV = 128, 64, 64
E_OP, E_PROC, E_MA = 320000, 320000, 16000
D_EDGE = 16

NC, NS, L = 2, 16, 16
K_BIG = 40
K_SMALL = 32
IB = 8
NSLOT_G = 2
NSLOT_O = 2
RING = 2

f32 = jnp.float32


def _mm_kernel(x_ref, w_ref, b_ref, o_ref):
    o_ref[...] = (
        jnp.dot(x_ref[...], w_ref[...], preferred_element_type=f32)
        + b_ref[...]
    )


def _proj_small_kernel(x_m_ref, w_m_ref, b_m_ref, x_a_ref, w_a_ref, b_a_ref,
                       ea_ref, we_ref, o_m_ref, o_a_ref, o_ea_ref):
    @pl.when(pl.program_id(0) == 0)
    def _():
        o_m_ref[...] = (
            jnp.dot(x_m_ref[...], w_m_ref[...], preferred_element_type=f32)
            + b_m_ref[...]
        )
        o_a_ref[...] = (
            jnp.dot(x_a_ref[...], w_a_ref[...], preferred_element_type=f32)
            + b_a_ref[...]
        )
    o_ea_ref[...] = jnp.dot(ea_ref[...], we_ref[...],
                            preferred_element_type=f32)


def _proj_small(x_m, w_m, b_m, x_a, w_a, b_a, ea, we):
    gb = 8000
    return pl.pallas_call(
        _proj_small_kernel,
        grid=(E_PROC // gb,),
        in_specs=[
            pl.BlockSpec((N_M, C_M), lambda i: (0, 0)),
            pl.BlockSpec((C_M, 2 * C_M), lambda i: (0, 0)),
            pl.BlockSpec((1, 2 * C_M), lambda i: (0, 0)),
            pl.BlockSpec((N_AGV, C_M), lambda i: (0, 0)),
            pl.BlockSpec((C_M, C_M), lambda i: (0, 0)),
            pl.BlockSpec((1, C_M), lambda i: (0, 0)),
            pl.BlockSpec((gb, D_EDGE), lambda i: (i, 0)),
            pl.BlockSpec((D_EDGE, C_M), lambda i: (0, 0)),
        ],
        out_specs=[
            pl.BlockSpec((N_M, 2 * C_M), lambda i: (0, 0)),
            pl.BlockSpec((N_AGV, C_M), lambda i: (0, 0)),
            pl.BlockSpec((gb, C_M), lambda i: (i, 0)),
        ],
        out_shape=(
            jax.ShapeDtypeStruct((N_M, 2 * C_M), f32),
            jax.ShapeDtypeStruct((N_AGV, C_M), f32),
            jax.ShapeDtypeStruct((E_PROC, C_M), f32),
        ),
    )(x_m, w_m, b_m.reshape(1, -1), x_a, w_a, b_a.reshape(1, -1), ea, we)


def _matmul_bias(x, w, b, block_m):
    m, k = x.shape
    n = w.shape[1]
    grid = m // block_m
    assert block_m * grid == m
    return pl.pallas_call(
        _mm_kernel,
        grid=(grid,),
        in_specs=[
            pl.BlockSpec((block_m, k), lambda i: (i, 0)),
            pl.BlockSpec((k, n), lambda i: (0, 0)),
            pl.BlockSpec((1, n), lambda i: (0, 0)),
        ],
        out_specs=pl.BlockSpec((block_m, n), lambda i: (i, 0)),
        out_shape=jax.ShapeDtypeStruct((m, n), f32),
    )(x, w, b.reshape(1, n))


def _zero_fill(buf, rows, width):
    z = jnp.zeros((L,), f32)

    def body(e, _):
        for j in range(width // L):
            buf[e, pl.ds(L * j, L)] = z
        return _

    lax.fori_loop(0, rows, body, None)


def _run_chunks(eidx, xl_hbm, xr_hbm, ea_hbm, att_vs, acc, c_dim, ke,
                k0, nk, total,
                sidx_ib, didx_ib, xl_s, xr_s, ea_s, out_s, sems, sems_o,
                lane0):
    nj = c_dim // L

    def blk(i):
        return jnp.minimum(k0 + (i // IB) * IB, total - IB)

    def fetch_blk(i):
        r0 = ((i // IB) % 2) * IB
        b0 = blk(i)
        pltpu.sync_copy(eidx.at[0, pl.ds(b0, IB)],
                        sidx_ib.at[pl.ds(r0, IB)])
        pltpu.sync_copy(eidx.at[1, pl.ds(b0, IB)],
                        didx_ib.at[pl.ds(r0, IB)])

    def idx_row(ib, i):
        return ib.at[((i // IB) % 2) * IB + (k0 + i - blk(i))]

    def gathers(i, sl):
        pltpu.async_copy(xl_hbm.at[idx_row(sidx_ib, i)], xl_s[sl], sems[sl])
        pltpu.async_copy(xr_hbm.at[idx_row(didx_ib, i)], xr_s[sl], sems[sl])
        if ea_hbm is not None:
            pltpu.async_copy(ea_hbm.at[pl.ds((k0 + i) * ke, ke)],
                             ea_s[sl], sems[sl])

    def drain(i, sl):
        pltpu.make_async_copy(
            xl_hbm.at[idx_row(sidx_ib, i)], xl_s[sl], sems[sl]).wait()
        pltpu.make_async_copy(
            xr_hbm.at[idx_row(didx_ib, i)], xr_s[sl], sems[sl]).wait()
        if ea_hbm is not None:
            pltpu.make_async_copy(
                ea_hbm.at[pl.ds((k0 + i) * ke, ke)], ea_s[sl],
                sems[sl]).wait()

    def scatter(i, sl):
        pltpu.async_copy(out_s[sl], acc.at[idx_row(didx_ib, i)],
                         sems_o[sl], add=True)

    def drain_scatter(i, sl):
        pltpu.make_async_copy(out_s[sl], acc.at[idx_row(didx_ib, i)],
                              sems_o[sl]).wait()

    def compute(sl, so):
        xl_b, xr_b = xl_s[sl], xr_s[sl]
        out_b = out_s[so]
        ea_b = ea_s[sl] if ea_hbm is not None else None

        def edge_body(e2, _):
            for eo in range(2):
                e = 2 * e2 + eo
                p = jnp.zeros((L,), f32)
                rows = []
                for j in range(nj):
                    v = xl_b[e, pl.ds(L * j, L)]
                    t = v + xr_b[e, pl.ds(L * j, L)]
                    if ea_b is not None:
                        t = t + ea_b[e, pl.ds(L * j, L)]
                    t = jnp.maximum(t, 0.2 * t)
                    p = p + att_vs[j] * t
                    rows.append(v)
                s = jnp.sum(p)
                ex = jnp.exp(lax.broadcast_in_dim(s, (L,), ()))
                for j in range(nj):
                    out_b[e, pl.ds(L * j, L)] = ex * rows[j]
                out_b[e, pl.ds(c_dim, L)] = ex * lane0
            return _

        lax.fori_loop(0, ke // 2, edge_body, None)

    fetch_blk(0)
    for d in range(NSLOT_G - 1):
        gathers(d, d)

    def ring_body(kr, _):
        for b in range(RING):
            i = RING * kr + b

            def step():
                nxt = i + NSLOT_G - 1

                @pl.when(jnp.logical_and(nxt < nk, nxt % IB == 0))
                def _():
                    fetch_blk(nxt)

                @pl.when(nxt < nk)
                def _():
                    gathers(nxt, (b + NSLOT_G - 1) % NSLOT_G)

                drain(i, b % NSLOT_G)

                @pl.when(i >= NSLOT_O)
                def _():
                    drain_scatter(i - NSLOT_O, b % NSLOT_O)

                compute(b % NSLOT_G, b % NSLOT_O)
                scatter(i, b % NSLOT_O)

            if b == 0:
                step()
            else:
                pl.when(i < nk)(step)
        return _

    lax.fori_loop(0, (nk + RING - 1) // RING, ring_body, None)

    for sl in range(NSLOT_O):
        pend = (nk - 1) - (nk - 1 - sl) % NSLOT_O
        drain_scatter(pend, sl)


def _sc_edge_kernel(
    xl_pred, xr_pred, xl_succ, xr_succ, xl_proc, xr_proc, ea_proc,
    xl_ma, xr_ma, att_pred, att_succ, att_proc, att_ma,
    ei_pred, ei_succ, ei_proc, ei_ma,
    out_big, out_proc, out_ma,
    acc_big, acc_proc, acc_ma,
):
    c = lax.axis_index("c")
    s = lax.axis_index("s")
    wid = c * NS + s

    lane0 = (lax.broadcasted_iota(jnp.int32, (L,), 0) == 0).astype(f32)

    def body(sem_big):
        _sc_edge_phases(
            xl_pred, xr_pred, xl_succ, xr_succ, xl_proc, xr_proc, ea_proc,
            xl_ma, xr_ma, att_pred, att_succ, att_proc, att_ma,
            ei_pred, ei_succ, ei_proc, ei_ma,
            out_big, out_proc, out_ma, acc_big, acc_proc, acc_ma,
            c, s, wid, lane0, sem_big)
        rows = N_OP // NS
        r0 = rows * s
        pltpu.make_async_copy(acc_big.at[pl.ds(r0, rows)],
                              out_big.at[c, pl.ds(r0, rows)], sem_big).wait()

    pl.run_scoped(body, pltpu.SemaphoreType.DMA)


def _sc_edge_phases(
    xl_pred, xr_pred, xl_succ, xr_succ, xl_proc, xr_proc, ea_proc,
    xl_ma, xr_ma, att_pred, att_succ, att_proc, att_ma,
    ei_pred, ei_succ, ei_proc, ei_ma,
    out_big, out_proc, out_ma, acc_big, acc_proc, acc_ma,
    c, s, wid, lane0, sem_big,
):

    def zero_rows_with(out_b, ke, acc, row0, nrows):
        zfull, zrem = divmod(nrows, ke)
        for kk in range(zfull):
            pltpu.sync_copy(out_b, acc.at[pl.ds(row0 + ke * kk, ke)])
        if zrem:
            pltpu.sync_copy(out_b.at[pl.ds(0, zrem)],
                            acc.at[pl.ds(row0 + ke * zfull, zrem)])

    def big_phase(sidx_ib, didx_ib, xl0, xl1, xr0, xr1,
                  out0, out1, att_b,
                  sem0, sem1, semo0, semo1):
        xl_s, xr_s = (xl0, xl1), (xr0, xr1)
        out_s = (out0, out1)
        sems, sems_o = (sem0, sem1), (semo0, semo1)
        _zero_fill(out0, K_BIG, C_OP + L)
        rows = N_OP // NS
        r0 = rows * s
        zero_rows_with(out0, K_BIG, acc_big, r0, rows)
        plsc.subcore_barrier()

        total = E_OP // K_BIG
        base, extra = divmod(total, NS)
        nk = base + (s < extra).astype(jnp.int32)
        k0 = base * s + jnp.minimum(s, extra)

        @pl.when(c == 0)
        def _():
            pltpu.sync_copy(att_pred, att_b)
            att_vs = [att_b[pl.ds(L * j, L)] for j in range(C_OP // L)]
            _run_chunks(ei_pred, xl_pred, xr_pred, None, att_vs, acc_big,
                        C_OP, K_BIG, k0, nk, total,
                        sidx_ib, didx_ib, xl_s, xr_s, None,
                        out_s, sems, sems_o, lane0)

        @pl.when(c == 1)
        def _():
            pltpu.sync_copy(att_succ, att_b)
            att_vs = [att_b[pl.ds(L * j, L)] for j in range(C_OP // L)]
            _run_chunks(ei_succ, xl_succ, xr_succ, None, att_vs, acc_big,
                        C_OP, K_BIG, k0, nk, total,
                        sidx_ib, didx_ib, xl_s, xr_s, None,
                        out_s, sems, sems_o, lane0)

        plsc.subcore_barrier()
        pltpu.async_copy(acc_big.at[pl.ds(r0, rows)],
                         out_big.at[c, pl.ds(r0, rows)], sem_big)

    pl.run_scoped(
        big_phase,
        pltpu.VMEM((2 * IB, K_BIG), jnp.int32),
        pltpu.VMEM((2 * IB, K_BIG), jnp.int32),
        pltpu.VMEM((K_BIG, C_OP), f32),
        pltpu.VMEM((K_BIG, C_OP), f32),
        pltpu.VMEM((K_BIG, C_OP), f32),
        pltpu.VMEM((K_BIG, C_OP), f32),
        pltpu.VMEM((K_BIG, C_OP + L), f32),
        pltpu.VMEM((K_BIG, C_OP + L), f32),
        pltpu.VMEM((C_OP,), f32),
        pltpu.SemaphoreType.DMA,
        pltpu.SemaphoreType.DMA,
        pltpu.SemaphoreType.DMA,
        pltpu.SemaphoreType.DMA,
    )

    def small_phase(sidx_ib, didx_ib, xl0, xl1, xr0, xr1,
                    ea0, ea1, out0, out1, att_b,
                    sem0, sem1, semo0, semo1):
        xl_s, xr_s = (xl0, xl1), (xr0, xr1)
        ea_s, out_s = (ea0, ea1), (out0, out1)
        sems, sems_o = (sem0, sem1), (semo0, semo1)
        _zero_fill(out0, K_SMALL, C_M + L)

        @pl.when(s < 8)
        def _():
            zero_rows_with(out0, K_SMALL, acc_proc, 125 * s, 125)

        @pl.when(jnp.logical_and(c == 0, s < 4))
        def _():
            zero_rows_with(out0, K_SMALL, acc_ma, 125 * s, 125)

        plsc.subcore_barrier()

        pltpu.sync_copy(att_proc, att_b)
        att_vs = [att_b[pl.ds(L * j, L)] for j in range(C_M // L)]
        total_p = E_PROC // K_SMALL
        pbase, pextra = divmod(total_p, NC * NS)
        nk = pbase + (wid < pextra).astype(jnp.int32)
        k0 = pbase * wid + jnp.minimum(wid, pextra)
        _run_chunks(ei_proc, xl_proc, xr_proc, ea_proc, att_vs, acc_proc,
                    C_M, K_SMALL, k0, nk, total_p,
                    sidx_ib, didx_ib, xl_s, xr_s, ea_s,
                    out_s, sems, sems_o, lane0)

        @pl.when(c == 0)
        def _():
            pltpu.sync_copy(att_ma, att_b)
            att_vs2 = [att_b[pl.ds(L * j, L)] for j in range(C_M // L)]
            total_m = E_MA // K_SMALL
            mbase, mextra = divmod(total_m, NS)
            nk2 = mbase + (s < mextra).astype(jnp.int32)
            k02 = mbase * s + jnp.minimum(s, mextra)
            _run_chunks(ei_ma, xl_ma, xr_ma, None, att_vs2, acc_ma,
                        C_M, K_SMALL, k02, nk2, total_m,
                        sidx_ib, didx_ib, xl_s, xr_s, None,
                        out_s, sems, sems_o, lane0)

        plsc.subcore_barrier()

        @pl.when(s < 10)
        def _():
            pltpu.sync_copy(acc_proc.at[pl.ds(100 * s, 100)],
                            out_proc.at[c, pl.ds(100 * s, 100)])

        @pl.when(jnp.logical_and(c == 0, s < 10))
        def _():
            pltpu.sync_copy(acc_ma.at[pl.ds(50 * s, 50)],
                            out_ma.at[pl.ds(50 * s, 50)])

    pl.run_scoped(
        small_phase,
        pltpu.VMEM((2 * IB, K_SMALL), jnp.int32),
        pltpu.VMEM((2 * IB, K_SMALL), jnp.int32),
        pltpu.VMEM((K_SMALL, C_M), f32),
        pltpu.VMEM((K_SMALL, C_M), f32),
        pltpu.VMEM((K_SMALL, C_M), f32),
        pltpu.VMEM((K_SMALL, C_M), f32),
        pltpu.VMEM((K_SMALL, C_M), f32),
        pltpu.VMEM((K_SMALL, C_M), f32),
        pltpu.VMEM((K_SMALL, C_M + L), f32),
        pltpu.VMEM((K_SMALL, C_M + L), f32),
        pltpu.VMEM((C_M,), f32),
        pltpu.SemaphoreType.DMA,
        pltpu.SemaphoreType.DMA,
        pltpu.SemaphoreType.DMA,
        pltpu.SemaphoreType.DMA,
    )


_sc_edge_call = functools.partial(
    pl.kernel,
    out_type=(
        jax.ShapeDtypeStruct((NC, N_OP, C_OP + L), f32),
        jax.ShapeDtypeStruct((NC, N_M, C_M + L), f32),
        jax.ShapeDtypeStruct((N_AGV, C_M + L), f32),
    ),
    mesh=plsc.VectorSubcoreMesh(core_axis_name="c", subcore_axis_name="s"),
    compiler_params=pltpu.CompilerParams(
        use_tc_tiling_on_sc=False, needs_layout_passes=False),
    scratch_types=[
        pltpu.VMEM_SHARED((N_OP, C_OP + L), f32),
        pltpu.VMEM_SHARED((N_M, C_M + L), f32),
        pltpu.VMEM_SHARED((N_AGV, C_M + L), f32),
    ],
)(_sc_edge_kernel)


def _finalize_kernel(x_op_ref, x_m_ref, x_agv_ref, big_ref, proc_ref, ma_ref,
                     bp_ref, bs_ref, bm_ref, ba_ref,
                     o_op_ref, o_m_ref, o_agv_ref):
    accp = big_ref[0]
    accs = big_ref[1]
    o_op_ref[...] = (
        x_op_ref[...]
        + accp[:, :C_OP] / (accp[:, C_OP:C_OP + 1] + 1e-16) + bp_ref[...]
        + accs[:, :C_OP] / (accs[:, C_OP:C_OP + 1] + 1e-16) + bs_ref[...]
    )
    pr = proc_ref[0] + proc_ref[1]
    o_m_ref[...] = (
        x_m_ref[...] + pr[:, :C_M] / (pr[:, C_M:C_M + 1] + 1e-16)
        + bm_ref[...]
    )
    ma = ma_ref[...]
    o_agv_ref[...] = (
        x_agv_ref[...] + ma[:, :C_M] / (ma[:, C_M:C_M + 1] + 1e-16)
        + ba_ref[...]
    )


def _finalize(x_op, x_m, x_agv, big, proc, ma, bp, bs, bm, ba):
    return pl.pallas_call(
        _finalize_kernel,
        out_shape=(
            jax.ShapeDtypeStruct((N_OP, C_OP), f32),
            jax.ShapeDtypeStruct((N_M, C_M), f32),
            jax.ShapeDtypeStruct((N_AGV, C_AGV), f32),
        ),
    )(x_op, x_m, x_agv, big, proc, ma,
      bp.reshape(1, -1), bs.reshape(1, -1), bm.reshape(1, -1),
      ba.reshape(1, -1))


@jax.jit
def kernel(x_operation, x_machine, x_AGV, edge_index_pred, edge_index_succ,
           edge_index_proc, edge_index_ma, edge_attr_proc,
           Wl_pred, bl_pred, Wr_pred, br_pred, att_pred, bias_pred,
           Wl_succ, bl_succ, Wr_succ, br_succ, att_succ, bias_succ,
           Wl_proc, bl_proc, Wr_proc, br_proc, att_proc, We_proc, bias_proc,
           Wl_ma, bl_ma, Wr_ma, br_ma, att_ma, bias_ma):
    w_op = jnp.concatenate([Wl_pred, Wr_pred, Wl_succ, Wr_succ, Wl_proc], 1)
    b_op = jnp.concatenate([bl_pred, br_pred, bl_succ, br_succ, bl_proc])
    proj_op = _matmul_bias(x_operation, w_op, b_op, 1000)
    xl_pred = proj_op[:, 0:128]
    xr_pred = proj_op[:, 128:256]
    xl_succ = proj_op[:, 256:384]
    xr_succ = proj_op[:, 384:512]
    xl_proc = proj_op[:, 512:576]

    w_m = jnp.concatenate([Wr_proc, Wl_ma], 1)
    b_m = jnp.concatenate([br_proc, bl_ma])
    proj_m, xr_ma, ea_proc = _proj_small(
        x_machine, w_m, b_m, x_AGV, Wr_ma, br_ma, edge_attr_proc, We_proc)
    xr_proc = proj_m[:, 0:64]
    xl_ma = proj_m[:, 64:128]

    out_big, out_proc, out_ma = _sc_edge_call(
        xl_pred, xr_pred, xl_succ, xr_succ, xl_proc, xr_proc, ea_proc,
        xl_ma, xr_ma, att_pred, att_succ, att_proc, att_ma,
        edge_index_pred.reshape(2, E_OP // K_BIG, K_BIG),
        edge_index_succ.reshape(2, E_OP // K_BIG, K_BIG),
        edge_index_proc.reshape(2, E_PROC // K_SMALL, K_SMALL),
        edge_index_ma.reshape(2, E_MA // K_SMALL, K_SMALL))

    return _finalize(x_operation, x_machine, x_AGV, out_big, out_proc, out_ma,
                     bias_pred, bias_succ, bias_proc, bias_ma)
